# Initial kernel scaffold; baseline (speedup 1.0000x reference)
import jax
import jax.numpy as jnp
from jax.experimental import pallas as pl


def kernel(photo_one_hop, user_emb, item_emb, image_w, text_w, Q, K, V, W_onehop, W_mul1, W_mul2, edge_index, edge_weight):
    raise NotImplementedError("write your pallas kernel here")


# speedup vs baseline: 1.0119x; 1.0119x over previous
import functools

import jax
import jax.numpy as jnp
from jax.experimental import pallas as pl
from jax.experimental.pallas import tpu as pltpu

NUM_USERS = 25000
NUM_ITEMS = 25000
HID = 64
N_LAYERS = 3
HIST = 20
N_NODES = NUM_USERS + NUM_ITEMS
N_EDGES = 800000
NH = 4
ATT = 16


def _mean4_kernel(a_ref, b_ref, c_ref, d_ref, o_ref):
    o_ref[...] = 0.25 * (a_ref[...] + b_ref[...] + c_ref[...] + d_ref[...])


def _mean4(a, b, c, d):
    n = a.shape[0]
    blk = 5000
    grid = (n // blk,)
    spec = pl.BlockSpec((blk, HID), lambda i: (i, 0))
    return pl.pallas_call(
        _mean4_kernel,
        grid=grid,
        in_specs=[spec, spec, spec, spec],
        out_specs=spec,
        out_shape=jax.ShapeDtypeStruct((n, HID), jnp.float32),
    )(a, b, c, d)


def _transformer(query_input, action_list, Q, K, V):
    q = jnp.tensordot(query_input, Q, axes=([-1], [0]))
    k = jnp.tensordot(action_list, K, axes=([-1], [0]))
    v = jnp.tensordot(action_list, V, axes=([-1], [0]))
    qs = jnp.stack(jnp.split(q, NH, axis=2))
    ks = jnp.stack(jnp.split(k, NH, axis=2))
    vs = jnp.stack(jnp.split(v, NH, axis=2))
    scores = jnp.matmul(qs, jnp.swapaxes(ks, -2, -1)) / 8.0
    att = jax.nn.softmax(scores, axis=-1)
    res = jnp.matmul(att, vs)
    res = jnp.transpose(res, (1, 2, 0, 3))
    return res.reshape(query_input.shape[0], NH * ATT)


def _spmm(edge_index, edge_weight, x):
    msgs = x[edge_index[1]] * edge_weight[:, None]
    return jax.ops.segment_sum(msgs, edge_index[0], num_segments=N_NODES)


def kernel(photo_one_hop, user_emb, item_emb, image_w, text_w, Q, K, V,
           W_onehop, W_mul1, W_mul2, edge_index, edge_weight):
    photo_one_hop_embeddings = user_emb[photo_one_hop]
    photo_query = item_emb.reshape(-1, 1, HID)
    mha = _transformer(photo_query, photo_one_hop_embeddings, Q, K, V)
    item_features_emb = jnp.concatenate([image_w, text_w], axis=-1)
    item_fea_hidden = item_features_emb @ W_mul1.T
    itea_fea_emb = item_fea_hidden @ W_mul2.T
    item_one_hop = mha @ W_onehop.T
    all_items = item_emb * itea_fea_emb + item_one_hop
    all_emb = jnp.concatenate([user_emb, all_items], axis=0)
    embs = [all_emb]
    for _ in range(N_LAYERS):
        all_emb = _spmm(edge_index, edge_weight, all_emb)
        embs.append(all_emb)
    light_out = _mean4(embs[0], embs[1], embs[2], embs[3])
    return light_out[:NUM_USERS], light_out[NUM_USERS:]


# speedup vs baseline: 1.8865x; 1.8643x over previous
import functools

import jax
import jax.numpy as jnp
from jax import lax
from jax.experimental import pallas as pl
from jax.experimental.pallas import tpu as pltpu
from jax.experimental.pallas import tpu_sc as plsc

NUM_USERS = 25000
NUM_ITEMS = 25000
HID = 64
N_LAYERS = 3
HIST = 20
N_NODES = NUM_USERS + NUM_ITEMS
N_EDGES = 800000
NH = 4
ATT = 16

NC = 2
NS = 16
LANES = 16

HALF = 25000
ACC_R = 25600
PAD = ACC_R - HALF
RPS = ACC_R // NS
XROWS = NC * ACC_R

CH = 256
JB = CH // 128
NCHUNK = 196
EPS_P = NCHUNK * CH
EDGES_P = NS * EPS_P


def _spmm_body(dst_hbm, src_hbm, w_hbm, x_hbm, zeros_hbm, out_hbm,
               acc_sh, dstv, srcv, wv, gsrc2d, lidx2d, rows, sem_g, sem_s):
    c = lax.axis_index("c")
    s = lax.axis_index("s")
    half_base = c * HALF

    pltpu.sync_copy(zeros_hbm.at[pl.ds(s * RPS, RPS)],
                    acc_sh.at[pl.ds(s * RPS, RPS)])
    plsc.subcore_barrier()

    ebase = s * EPS_P

    def chunk(i, carry):
        off = ebase + i * CH
        pltpu.sync_copy(dst_hbm.at[pl.ds(off, CH)], dstv)
        pltpu.sync_copy(src_hbm.at[pl.ds(off, CH)], srcv)
        pltpu.sync_copy(w_hbm.at[pl.ds(off, CH)], wv)

        def vec(g, carry2):
            j = g >> 3
            col = (g & 7) * LANES
            base = g * LANES
            d16 = dstv[pl.ds(base, LANES)]
            s16 = srcv[pl.ds(base, LANES)]
            w16 = wv[pl.ds(base, LANES)]
            loc = d16 - half_base
            m = (loc >= 0) & (loc < HALF)
            gsrc2d[j, pl.ds(col, LANES)] = s16 + jnp.where(
                s16 >= HALF, jnp.int32(PAD), jnp.int32(0))
            lidx2d[j, pl.ds(col, LANES)] = jnp.where(m, loc, loc & 8191)
            wv[pl.ds(base, LANES)] = jnp.where(m, w16, 0.0)
            return carry2

        lax.fori_loop(0, CH // LANES, vec, 0)

        descs = []
        for j in range(JB):
            descs.append(pltpu.async_copy(
                x_hbm.at[gsrc2d.at[j]], rows.at[pl.ds(j * 128, 128)], sem_g))
        for d in descs:
            d.wait()

        def scale(g, carry2):
            w16 = wv[pl.ds(g * LANES, LANES)]
            for l in range(LANES):
                e = g * LANES + l
                w = w16[l]
                for dd in range(HID // LANES):
                    sl = pl.ds(dd * LANES, LANES)
                    rows[e, sl] = rows[e, sl] * w
            return carry2

        lax.fori_loop(0, CH // LANES, scale, 0)

        descs2 = []
        for j in range(JB):
            descs2.append(pltpu.async_copy(
                rows.at[pl.ds(j * 128, 128)], acc_sh.at[lidx2d.at[j]],
                sem_s, add=True))
        for d in descs2:
            d.wait()
        return carry

    lax.fori_loop(0, NCHUNK, chunk, 0)

    plsc.subcore_barrier()
    pltpu.sync_copy(acc_sh.at[pl.ds(s * RPS, RPS)],
                    out_hbm.at[pl.ds(c * ACC_R + s * RPS, RPS)])


_spmm_call = pl.kernel(
    _spmm_body,
    out_type=jax.ShapeDtypeStruct((XROWS, HID), jnp.float32),
    mesh=plsc.VectorSubcoreMesh(core_axis_name="c", subcore_axis_name="s",
                                num_cores=NC, num_subcores=NS),
    scratch_types=[
        pltpu.VMEM_SHARED((ACC_R, HID), jnp.float32),
        pltpu.VMEM((CH,), jnp.int32),
        pltpu.VMEM((CH,), jnp.int32),
        pltpu.VMEM((CH,), jnp.float32),
        pltpu.VMEM((JB, 128), jnp.int32),
        pltpu.VMEM((JB, 128), jnp.int32),
        pltpu.VMEM((CH, HID), jnp.float32),
        pltpu.SemaphoreType.DMA,
        pltpu.SemaphoreType.DMA,
    ],
    compiler_params=pltpu.CompilerParams(use_tc_tiling_on_sc=False),
)


def _mean4_kernel(a_ref, b_ref, c_ref, d_ref, o_ref):
    o_ref[...] = 0.25 * (a_ref[...] + b_ref[...] + c_ref[...] + d_ref[...])


def _mean4_half(e0, e1, e2, e3, block_off):
    blk = 200
    in_spec = pl.BlockSpec((blk, HID), lambda i: (i + block_off, 0))
    out_spec = pl.BlockSpec((blk, HID), lambda i: (i, 0))
    return pl.pallas_call(
        _mean4_kernel,
        grid=(NUM_USERS // blk,),
        in_specs=[in_spec] * 4,
        out_specs=out_spec,
        out_shape=jax.ShapeDtypeStruct((NUM_USERS, HID), jnp.float32),
    )(e0, e1, e2, e3)


def _transformer(query_input, action_list, Q, K, V):
    q = jnp.tensordot(query_input, Q, axes=([-1], [0]))
    k = jnp.tensordot(action_list, K, axes=([-1], [0]))
    v = jnp.tensordot(action_list, V, axes=([-1], [0]))
    qs = jnp.stack(jnp.split(q, NH, axis=2))
    ks = jnp.stack(jnp.split(k, NH, axis=2))
    vs = jnp.stack(jnp.split(v, NH, axis=2))
    scores = jnp.matmul(qs, jnp.swapaxes(ks, -2, -1)) / 8.0
    att = jax.nn.softmax(scores, axis=-1)
    res = jnp.matmul(att, vs)
    res = jnp.transpose(res, (1, 2, 0, 3))
    return res.reshape(query_input.shape[0], NH * ATT)


def kernel(photo_one_hop, user_emb, item_emb, image_w, text_w, Q, K, V,
           W_onehop, W_mul1, W_mul2, edge_index, edge_weight):
    photo_one_hop_embeddings = user_emb[photo_one_hop]
    photo_query = item_emb.reshape(-1, 1, HID)
    mha = _transformer(photo_query, photo_one_hop_embeddings, Q, K, V)
    item_features_emb = jnp.concatenate([image_w, text_w], axis=-1)
    item_fea_hidden = item_features_emb @ W_mul1.T
    itea_fea_emb = item_fea_hidden @ W_mul2.T
    item_one_hop = mha @ W_onehop.T
    all_items = item_emb * itea_fea_emb + item_one_hop

    zpad = jnp.zeros((PAD, HID), jnp.float32)
    x = jnp.concatenate([user_emb, zpad, all_items, zpad], axis=0)

    pad_n = EDGES_P - N_EDGES
    fill = (jnp.arange(pad_n, dtype=jnp.int32) * 37) % N_NODES
    dst_p = jnp.concatenate([edge_index[0], fill])
    src_p = jnp.concatenate([edge_index[1], fill])
    w_p = jnp.concatenate([edge_weight, jnp.zeros((pad_n,), jnp.float32)])
    zeros_acc = jnp.zeros((ACC_R, HID), jnp.float32)

    embs = [x]
    for _ in range(N_LAYERS):
        x = _spmm_call(dst_p, src_p, w_p, x, zeros_acc)
        embs.append(x)

    users_emb = _mean4_half(embs[0], embs[1], embs[2], embs[3], 0)
    items_emb = _mean4_half(embs[0], embs[1], embs[2], embs[3], ACC_R // 200)
    return users_emb, items_emb


# speedup vs baseline: 3.9104x; 2.0728x over previous
import functools

import jax
import jax.numpy as jnp
from jax import lax
from jax.experimental import pallas as pl
from jax.experimental.pallas import tpu as pltpu
from jax.experimental.pallas import tpu_sc as plsc

NUM_USERS = 25000
NUM_ITEMS = 25000
HID = 64
N_LAYERS = 3
HIST = 20
N_NODES = NUM_USERS + NUM_ITEMS
N_EDGES = 800000
NH = 4
ATT = 16

NC = 2
NS = 16
LANES = 16

HALF = 25000
ACC_R = 25600
PAD = ACC_R - HALF
RPS = ACC_R // NS
XROWS = NC * ACC_R

CH = 128
NCHUNK = 392
EPS_P = NCHUNK * CH
EDGES_P = NS * EPS_P
REC = 3 * CH


def _spmm_body(pack_hbm, x_hbm, zeros_hbm, out_hbm,
               acc_sh, packb, gsrc2, lidx2, wbuf, rows2,
               sem_p0, sem_p1, sem_g0, sem_g1, sem_s0, sem_s1):
    c = lax.axis_index("c")
    s = lax.axis_index("s")
    half_base = c * HALF
    sem_p = (sem_p0, sem_p1)
    sem_g = (sem_g0, sem_g1)
    sem_s = (sem_s0, sem_s1)

    pltpu.sync_copy(zeros_hbm.at[pl.ds(s * RPS, RPS)],
                    acc_sh.at[pl.ds(s * RPS, RPS)])
    plsc.subcore_barrier()

    rbase = s * NCHUNK

    def fire_pack(i, b):
        pltpu.async_copy(pack_hbm.at[pl.ds((rbase + i) * REC, REC)],
                         packb.at[b], sem_p[b])

    def vec_pass(b):
        for g in range(CH // LANES):
            o = g * LANES
            d16 = packb[b, pl.ds(o, LANES)]
            s16 = packb[b, pl.ds(CH + o, LANES)]
            w16 = plsc.bitcast(packb[b, pl.ds(2 * CH + o, LANES)],
                               jnp.float32)
            loc = d16 - half_base
            m = (loc >= 0) & (loc < HALF)
            gsrc2[b, pl.ds(o, LANES)] = s16 + jnp.where(
                s16 >= HALF, jnp.int32(PAD), jnp.int32(0))
            lidx2[b, pl.ds(o, LANES)] = jnp.where(m, loc, loc & 8191)
            wbuf[b, pl.ds(o, LANES)] = jnp.where(m, w16, 0.0)

    def scale_pass(b):
        for g in range(CH // LANES):
            w16 = wbuf[b, pl.ds(g * LANES, LANES)]
            for l in range(LANES):
                e = g * LANES + l
                w = w16[l]
                for dd in range(HID // LANES):
                    sl = pl.ds(dd * LANES, LANES)
                    rows2[b, e, sl] = rows2[b, e, sl] * w

    fire_pack(0, 0)
    fire_pack(1, 1)

    def half_step(i, b):
        @pl.when(jnp.logical_and(i >= 2, i <= NCHUNK + 1))
        def _():
            pltpu.make_async_copy(
                rows2.at[b], acc_sh.at[pl.ds(0, CH)], sem_s[b]).wait()

        @pl.when(i < NCHUNK)
        def _():
            pltpu.make_async_copy(
                pack_hbm.at[pl.ds((rbase + i) * REC, REC)], packb.at[b],
                sem_p[b]).wait()
            vec_pass(b)
            pltpu.async_copy(x_hbm.at[gsrc2.at[b]], rows2.at[b], sem_g[b])

        nb = 1 - b

        @pl.when(jnp.logical_and(i >= 1, i <= NCHUNK))
        def _():
            pltpu.make_async_copy(
                x_hbm.at[gsrc2.at[nb]], rows2.at[nb], sem_g[nb]).wait()
            scale_pass(nb)
            pltpu.async_copy(rows2.at[nb], acc_sh.at[lidx2.at[nb]],
                             sem_s[nb], add=True)

        @pl.when(i + 2 < NCHUNK)
        def _():
            fire_pack(i + 2, b)

    def step(g, carry):
        half_step(2 * g, 0)
        half_step(2 * g + 1, 1)
        return carry

    lax.fori_loop(0, (NCHUNK + 2) // 2, step, 0)

    plsc.subcore_barrier()
    pltpu.sync_copy(acc_sh.at[pl.ds(s * RPS, RPS)],
                    out_hbm.at[pl.ds(c * ACC_R + s * RPS, RPS)])


_spmm_call = pl.kernel(
    _spmm_body,
    out_type=jax.ShapeDtypeStruct((XROWS, HID), jnp.float32),
    mesh=plsc.VectorSubcoreMesh(core_axis_name="c", subcore_axis_name="s",
                                num_cores=NC, num_subcores=NS),
    scratch_types=[
        pltpu.VMEM_SHARED((ACC_R, HID), jnp.float32),
        pltpu.VMEM((2, REC), jnp.int32),
        pltpu.VMEM((2, CH), jnp.int32),
        pltpu.VMEM((2, CH), jnp.int32),
        pltpu.VMEM((2, CH), jnp.float32),
        pltpu.VMEM((2, CH, HID), jnp.float32),
        pltpu.SemaphoreType.DMA,
        pltpu.SemaphoreType.DMA,
        pltpu.SemaphoreType.DMA,
        pltpu.SemaphoreType.DMA,
        pltpu.SemaphoreType.DMA,
        pltpu.SemaphoreType.DMA,
    ],
    compiler_params=pltpu.CompilerParams(use_tc_tiling_on_sc=False,
                                         needs_layout_passes=False),
)


def _mean4_kernel(a_ref, b_ref, c_ref, d_ref, o_ref):
    o_ref[...] = 0.25 * (a_ref[...] + b_ref[...] + c_ref[...] + d_ref[...])


def _mean4_half(e0, e1, e2, e3, block_off):
    blk = 200
    in_spec = pl.BlockSpec((blk, HID), lambda i: (i + block_off, 0))
    out_spec = pl.BlockSpec((blk, HID), lambda i: (i, 0))
    return pl.pallas_call(
        _mean4_kernel,
        grid=(NUM_USERS // blk,),
        in_specs=[in_spec] * 4,
        out_specs=out_spec,
        out_shape=jax.ShapeDtypeStruct((NUM_USERS, HID), jnp.float32),
    )(e0, e1, e2, e3)


def _transformer(query_input, action_list, Q, K, V):
    q = jnp.tensordot(query_input, Q, axes=([-1], [0]))
    k = jnp.tensordot(action_list, K, axes=([-1], [0]))
    v = jnp.tensordot(action_list, V, axes=([-1], [0]))
    qs = jnp.stack(jnp.split(q, NH, axis=2))
    ks = jnp.stack(jnp.split(k, NH, axis=2))
    vs = jnp.stack(jnp.split(v, NH, axis=2))
    scores = jnp.matmul(qs, jnp.swapaxes(ks, -2, -1)) / 8.0
    att = jax.nn.softmax(scores, axis=-1)
    res = jnp.matmul(att, vs)
    res = jnp.transpose(res, (1, 2, 0, 3))
    return res.reshape(query_input.shape[0], NH * ATT)


def kernel(photo_one_hop, user_emb, item_emb, image_w, text_w, Q, K, V,
           W_onehop, W_mul1, W_mul2, edge_index, edge_weight):
    photo_one_hop_embeddings = user_emb[photo_one_hop]
    photo_query = item_emb.reshape(-1, 1, HID)
    mha = _transformer(photo_query, photo_one_hop_embeddings, Q, K, V)
    item_features_emb = jnp.concatenate([image_w, text_w], axis=-1)
    item_fea_hidden = item_features_emb @ W_mul1.T
    itea_fea_emb = item_fea_hidden @ W_mul2.T
    item_one_hop = mha @ W_onehop.T
    all_items = item_emb * itea_fea_emb + item_one_hop

    zpad = jnp.zeros((PAD, HID), jnp.float32)
    x = jnp.concatenate([user_emb, zpad, all_items, zpad], axis=0)

    pad_n = EDGES_P - N_EDGES
    fill = (jnp.arange(pad_n, dtype=jnp.int32) * 37) % N_NODES
    dst_p = jnp.concatenate([edge_index[0], fill])
    src_p = jnp.concatenate([edge_index[1], fill])
    w_p = jnp.concatenate([edge_weight, jnp.zeros((pad_n,), jnp.float32)])
    w_bits = jax.lax.bitcast_convert_type(w_p, jnp.int32)
    pack = jnp.stack([dst_p.reshape(-1, CH), src_p.reshape(-1, CH),
                      w_bits.reshape(-1, CH)], axis=1).reshape(-1)
    zeros_acc = jnp.zeros((ACC_R, HID), jnp.float32)

    embs = [x]
    for _ in range(N_LAYERS):
        x = _spmm_call(pack, x, zeros_acc)
        embs.append(x)

    users_emb = _mean4_half(embs[0], embs[1], embs[2], embs[3], 0)
    items_emb = _mean4_half(embs[0], embs[1], embs[2], embs[3], ACC_R // 200)
    return users_emb, items_emb


# speedup vs baseline: 4.0815x; 1.0438x over previous
import functools

import jax
import jax.numpy as jnp
from jax import lax
from jax.experimental import pallas as pl
from jax.experimental.pallas import tpu as pltpu
from jax.experimental.pallas import tpu_sc as plsc

NUM_USERS = 25000
NUM_ITEMS = 25000
HID = 64
N_LAYERS = 3
HIST = 20
N_NODES = NUM_USERS + NUM_ITEMS
N_EDGES = 800000
NH = 4
ATT = 16

NC = 2
NS = 16
LANES = 16

HALF = 25000
ACC_R = 25600
PAD = ACC_R - HALF
RPS = ACC_R // NS
XROWS = NC * ACC_R

CH = 128
NCHUNK = 392
EPS_P = NCHUNK * CH
EDGES_P = NS * EPS_P
REC = 3 * CH


def _spmm_body(pack_hbm, x_hbm, zeros_hbm, out_hbm,
               acc_sh, packb, gsrc2, lidx2, wbuf, rows2,
               sem_p0, sem_p1, sem_g0, sem_g1, sem_s0, sem_s1):
    c = lax.axis_index("c")
    s = lax.axis_index("s")
    half_base = c * HALF
    sem_p = (sem_p0, sem_p1)
    sem_g = (sem_g0, sem_g1)
    sem_s = (sem_s0, sem_s1)

    pltpu.sync_copy(zeros_hbm.at[pl.ds(s * RPS, RPS)],
                    acc_sh.at[pl.ds(s * RPS, RPS)])
    plsc.subcore_barrier()

    rbase = s * NCHUNK

    def fire_pack(i, b):
        pltpu.async_copy(pack_hbm.at[pl.ds((rbase + i) * REC, REC)],
                         packb.at[b], sem_p[b])

    def vec_pass(b):
        for g in range(CH // LANES):
            o = g * LANES
            d16 = packb[b, pl.ds(o, LANES)]
            s16 = packb[b, pl.ds(CH + o, LANES)]
            w16 = plsc.bitcast(packb[b, pl.ds(2 * CH + o, LANES)],
                               jnp.float32)
            loc = d16 - half_base
            m = (loc >= 0) & (loc < HALF)
            gsrc2[b, pl.ds(o, LANES)] = s16 + jnp.where(
                s16 >= HALF, jnp.int32(PAD), jnp.int32(0))
            lidx2[b, pl.ds(o, LANES)] = jnp.where(m, loc, loc & 8191)
            wbuf[b, pl.ds(o, LANES)] = jnp.where(m, w16, 0.0)

    def scale_pass(b):
        for g in range(CH // LANES):
            w16 = wbuf[b, pl.ds(g * LANES, LANES)]
            for l in range(LANES):
                e = g * LANES + l
                w = w16[l]
                for dd in range(HID // LANES):
                    sl = pl.ds(dd * LANES, LANES)
                    rows2[b, e, sl] = rows2[b, e, sl] * w

    fire_pack(0, 0)
    fire_pack(1, 1)

    def half_step(i, b):
        @pl.when(jnp.logical_and(i >= 2, i <= NCHUNK + 1))
        def _():
            pltpu.make_async_copy(
                rows2.at[b], acc_sh.at[pl.ds(0, CH)], sem_s[b]).wait()

        @pl.when(i < NCHUNK)
        def _():
            pltpu.make_async_copy(
                pack_hbm.at[pl.ds((rbase + i) * REC, REC)], packb.at[b],
                sem_p[b]).wait()
            vec_pass(b)
            pltpu.async_copy(x_hbm.at[gsrc2.at[b]], rows2.at[b], sem_g[b])

        nb = 1 - b

        @pl.when(jnp.logical_and(i >= 1, i <= NCHUNK))
        def _():
            pltpu.make_async_copy(
                x_hbm.at[gsrc2.at[nb]], rows2.at[nb], sem_g[nb]).wait()
            scale_pass(nb)
            pltpu.async_copy(rows2.at[nb], acc_sh.at[lidx2.at[nb]],
                             sem_s[nb], add=True)

        @pl.when(i + 2 < NCHUNK)
        def _():
            fire_pack(i + 2, b)

    def step(g, carry):
        half_step(2 * g, 0)
        half_step(2 * g + 1, 1)
        return carry

    lax.fori_loop(0, (NCHUNK + 2) // 2, step, 0)

    plsc.subcore_barrier()
    pltpu.sync_copy(acc_sh.at[pl.ds(s * RPS, RPS)],
                    out_hbm.at[pl.ds(c * ACC_R + s * RPS, RPS)])


_spmm_call = pl.kernel(
    _spmm_body,
    out_type=jax.ShapeDtypeStruct((XROWS, HID), jnp.float32),
    mesh=plsc.VectorSubcoreMesh(core_axis_name="c", subcore_axis_name="s",
                                num_cores=NC, num_subcores=NS),
    scratch_types=[
        pltpu.VMEM_SHARED((ACC_R, HID), jnp.float32),
        pltpu.VMEM((2, REC), jnp.int32),
        pltpu.VMEM((2, CH), jnp.int32),
        pltpu.VMEM((2, CH), jnp.int32),
        pltpu.VMEM((2, CH), jnp.float32),
        pltpu.VMEM((2, CH, HID), jnp.float32),
        pltpu.SemaphoreType.DMA,
        pltpu.SemaphoreType.DMA,
        pltpu.SemaphoreType.DMA,
        pltpu.SemaphoreType.DMA,
        pltpu.SemaphoreType.DMA,
        pltpu.SemaphoreType.DMA,
    ],
    compiler_params=pltpu.CompilerParams(use_tc_tiling_on_sc=False,
                                         needs_layout_passes=False),
)


CG = 128
NIDX = NUM_ITEMS * HIST
NCH_G = 123
NW = NC * NS
GROWS = NW * NCH_G * CG


def _gather_body(idx_hbm, kv_hbm, out_hbm, idxb, rows2,
                 sem_p0, sem_p1, sem_g0, sem_g1, sem_w0, sem_w1):
    c = lax.axis_index("c")
    s = lax.axis_index("s")
    w = s * NC + c
    sem_p = (sem_p0, sem_p1)
    sem_g = (sem_g0, sem_g1)
    sem_w = (sem_w0, sem_w1)
    cbase = w * NCH_G

    def fire_idx(i, b):
        pltpu.async_copy(idx_hbm.at[pl.ds((cbase + i) * CG, CG)],
                         idxb.at[b], sem_p[b])

    fire_idx(0, 0)
    fire_idx(1, 1)

    def half_step(i, b):
        @pl.when(jnp.logical_and(i >= 2, i <= NCH_G + 1))
        def _():
            pltpu.make_async_copy(
                rows2.at[b], out_hbm.at[pl.ds(0, CG)], sem_w[b]).wait()

        @pl.when(i < NCH_G)
        def _():
            pltpu.make_async_copy(
                idx_hbm.at[pl.ds((cbase + i) * CG, CG)], idxb.at[b],
                sem_p[b]).wait()
            pltpu.async_copy(kv_hbm.at[idxb.at[b]], rows2.at[b], sem_g[b])

        nb = 1 - b

        @pl.when(jnp.logical_and(i >= 1, i <= NCH_G))
        def _():
            pltpu.make_async_copy(
                kv_hbm.at[idxb.at[nb]], rows2.at[nb], sem_g[nb]).wait()
            pltpu.async_copy(
                rows2.at[nb],
                out_hbm.at[pl.ds((cbase + i - 1) * CG, CG)], sem_w[nb])

        @pl.when(jnp.logical_and(i >= 1, i + 1 < NCH_G))
        def _():
            fire_idx(i + 1, nb)

    def step(g, carry):
        half_step(2 * g, 0)
        half_step(2 * g + 1, 1)
        return carry

    lax.fori_loop(0, (NCH_G + 3) // 2, step, 0)


_gather_call = pl.kernel(
    _gather_body,
    out_type=jax.ShapeDtypeStruct((GROWS, 2 * HID), jnp.float32),
    mesh=plsc.VectorSubcoreMesh(core_axis_name="c", subcore_axis_name="s",
                                num_cores=NC, num_subcores=NS),
    scratch_types=[
        pltpu.VMEM((2, CG), jnp.int32),
        pltpu.VMEM((2, CG, 2 * HID), jnp.float32),
        pltpu.SemaphoreType.DMA,
        pltpu.SemaphoreType.DMA,
        pltpu.SemaphoreType.DMA,
        pltpu.SemaphoreType.DMA,
        pltpu.SemaphoreType.DMA,
        pltpu.SemaphoreType.DMA,
    ],
)


B1 = 1000
B2 = 200


def _feat_kernel(img_ref, txt_ref, w1i_ref, w1t_ref, w2_ref, o_ref):
    h = (jnp.dot(img_ref[...], w1i_ref[...],
                 preferred_element_type=jnp.float32,
                 precision=lax.Precision.HIGHEST)
         + jnp.dot(txt_ref[...], w1t_ref[...],
                   preferred_element_type=jnp.float32,
                   precision=lax.Precision.HIGHEST))
    o_ref[...] = jnp.dot(h, w2_ref[...], preferred_element_type=jnp.float32,
                         precision=lax.Precision.HIGHEST)


def _feat_call(image_w, text_w, W_mul1, W_mul2):
    w1i = W_mul1[:, :4096].T
    w1t = W_mul1[:, 4096:].T
    w2 = W_mul2.T
    return pl.pallas_call(
        _feat_kernel,
        grid=(NUM_ITEMS // B1,),
        in_specs=[
            pl.BlockSpec((B1, 4096), lambda i: (i, 0)),
            pl.BlockSpec((B1, 384), lambda i: (i, 0)),
            pl.BlockSpec((4096, 256), lambda i: (0, 0)),
            pl.BlockSpec((384, 256), lambda i: (0, 0)),
            pl.BlockSpec((256, HID), lambda i: (0, 0)),
        ],
        out_specs=pl.BlockSpec((B1, HID), lambda i: (i, 0)),
        out_shape=jax.ShapeDtypeStruct((NUM_ITEMS, HID), jnp.float32),
    )(image_w, text_w, w1i, w1t, w2)


def _att_kernel(kv_ref, item_ref, qw_ref, m_ref, ow_ref, itea_ref, o_ref):
    kv = kv_ref[...]
    k = kv[:, :HID]
    v = kv[:, HID:]
    item = item_ref[...]
    q = jnp.dot(item, qw_ref[...], preferred_element_type=jnp.float32,
                precision=lax.Precision.HIGHEST)
    k3 = k.reshape(B2, HIST, HID)
    p = q[:, None, :] * k3
    m = m_ref[...]
    scores = jnp.dot(p.reshape(B2 * HIST, HID), m,
                     preferred_element_type=jnp.float32,
                     precision=lax.Precision.HIGHEST) / 8.0
    s3 = scores.reshape(B2, HIST, NH)
    mx = jnp.max(s3, axis=1, keepdims=True)
    ex = jnp.exp(s3 - mx)
    att = ex / jnp.sum(ex, axis=1, keepdims=True)
    attb = jnp.dot(att.reshape(B2 * HIST, NH), m.T,
                   preferred_element_type=jnp.float32,
                   precision=lax.Precision.HIGHEST)
    res = jnp.sum((attb * v).reshape(B2, HIST, HID), axis=1)
    oneh = jnp.dot(res, ow_ref[...], preferred_element_type=jnp.float32,
                   precision=lax.Precision.HIGHEST)
    o_ref[...] = item * itea_ref[...] + oneh


def _att_call(ph_kv, item_emb, Q, W_onehop, itea):
    m = (jnp.arange(HID, dtype=jnp.int32)[:, None] // ATT
         == jnp.arange(NH, dtype=jnp.int32)[None, :]).astype(jnp.float32)
    return pl.pallas_call(
        _att_kernel,
        grid=(NUM_ITEMS // B2,),
        in_specs=[
            pl.BlockSpec((B2 * HIST, 2 * HID), lambda i: (i, 0)),
            pl.BlockSpec((B2, HID), lambda i: (i, 0)),
            pl.BlockSpec((HID, HID), lambda i: (0, 0)),
            pl.BlockSpec((HID, NH), lambda i: (0, 0)),
            pl.BlockSpec((HID, HID), lambda i: (0, 0)),
            pl.BlockSpec((B2, HID), lambda i: (i, 0)),
        ],
        out_specs=pl.BlockSpec((B2, HID), lambda i: (i, 0)),
        out_shape=jax.ShapeDtypeStruct((NUM_ITEMS, HID), jnp.float32),
    )(ph_kv, item_emb, Q, m, W_onehop.T, itea)


def _mean4_kernel(a_ref, b_ref, c_ref, d_ref, o_ref):
    o_ref[...] = 0.25 * (a_ref[...] + b_ref[...] + c_ref[...] + d_ref[...])


def _mean4_half(e0, e1, e2, e3, block_off):
    blk = 200
    in_spec = pl.BlockSpec((blk, HID), lambda i: (i + block_off, 0))
    out_spec = pl.BlockSpec((blk, HID), lambda i: (i, 0))
    return pl.pallas_call(
        _mean4_kernel,
        grid=(NUM_USERS // blk,),
        in_specs=[in_spec] * 4,
        out_specs=out_spec,
        out_shape=jax.ShapeDtypeStruct((NUM_USERS, HID), jnp.float32),
    )(e0, e1, e2, e3)


def kernel(photo_one_hop, user_emb, item_emb, image_w, text_w, Q, K, V,
           W_onehop, W_mul1, W_mul2, edge_index, edge_weight):
    kv = jnp.concatenate(
        [jnp.dot(user_emb, K, precision=lax.Precision.HIGHEST),
         jnp.dot(user_emb, V, precision=lax.Precision.HIGHEST)],
        axis=1)
    pad_g = GROWS - NIDX
    gfill = (jnp.arange(pad_g, dtype=jnp.int32) * 13) % NUM_USERS
    gidx = jnp.concatenate([photo_one_hop.reshape(-1), gfill])
    ph_kv = _gather_call(gidx, kv)

    itea_fea_emb = _feat_call(image_w, text_w, W_mul1, W_mul2)
    all_items = _att_call(ph_kv, item_emb, Q, W_onehop, itea_fea_emb)

    zpad = jnp.zeros((PAD, HID), jnp.float32)
    x = jnp.concatenate([user_emb, zpad, all_items, zpad], axis=0)

    pad_n = EDGES_P - N_EDGES
    fill = (jnp.arange(pad_n, dtype=jnp.int32) * 37) % N_NODES
    dst_p = jnp.concatenate([edge_index[0], fill])
    src_p = jnp.concatenate([edge_index[1], fill])
    w_p = jnp.concatenate([edge_weight, jnp.zeros((pad_n,), jnp.float32)])
    w_bits = jax.lax.bitcast_convert_type(w_p, jnp.int32)
    pack = jnp.stack([dst_p.reshape(-1, CH), src_p.reshape(-1, CH),
                      w_bits.reshape(-1, CH)], axis=1).reshape(-1)
    zeros_acc = jnp.zeros((ACC_R, HID), jnp.float32)

    embs = [x]
    for _ in range(N_LAYERS):
        x = _spmm_call(pack, x, zeros_acc)
        embs.append(x)

    users_emb = _mean4_half(embs[0], embs[1], embs[2], embs[3], 0)
    items_emb = _mean4_half(embs[0], embs[1], embs[2], embs[3], ACC_R // 200)
    return users_emb, items_emb


# speedup vs baseline: 5.0753x; 1.2435x over previous
import functools

import jax
import jax.numpy as jnp
from jax import lax
from jax.experimental import pallas as pl
from jax.experimental.pallas import tpu as pltpu
from jax.experimental.pallas import tpu_sc as plsc

NUM_USERS = 25000
NUM_ITEMS = 25000
HID = 64
N_LAYERS = 3
HIST = 20
N_NODES = NUM_USERS + NUM_ITEMS
N_EDGES = 800000
NH = 4
ATT = 16

NC = 2
NS = 16
LANES = 16

HALF = 25000
ACC_R = 25600
PAD = ACC_R - HALF
RPS = ACC_R // NS
XROWS = NC * ACC_R

CH = 128
NCHUNK = 392
EPS_P = NCHUNK * CH
EDGES_P = NS * EPS_P
REC = 3 * CH


def _spmm_body(pack_hbm, x_hbm, zeros_hbm, out_hbm,
               acc_sh, packb, gsrc2, lidx2, wbuf, rows2,
               sem_p0, sem_p1, sem_g0, sem_g1, sem_s0, sem_s1):
    c = lax.axis_index("c")
    s = lax.axis_index("s")
    half_base = c * HALF
    sem_p = (sem_p0, sem_p1)
    sem_g = (sem_g0, sem_g1)
    sem_s = (sem_s0, sem_s1)

    pltpu.sync_copy(zeros_hbm.at[pl.ds(s * RPS, RPS)],
                    acc_sh.at[pl.ds(s * RPS, RPS)])
    plsc.subcore_barrier()

    rbase = s * NCHUNK

    def fire_pack(i, b):
        pltpu.async_copy(pack_hbm.at[pl.ds((rbase + i) * REC, REC)],
                         packb.at[b], sem_p[b])

    def vec_pass(b):
        for g in range(CH // LANES):
            o = g * LANES
            d16 = packb[b, pl.ds(o, LANES)]
            s16 = packb[b, pl.ds(CH + o, LANES)]
            w16 = plsc.bitcast(packb[b, pl.ds(2 * CH + o, LANES)],
                               jnp.float32)
            loc = d16 - half_base
            m = (loc >= 0) & (loc < HALF)
            gsrc2[b, pl.ds(o, LANES)] = s16 + jnp.where(
                s16 >= HALF, jnp.int32(PAD), jnp.int32(0))
            lidx2[b, pl.ds(o, LANES)] = jnp.where(m, loc, loc & 8191)
            wbuf[b, pl.ds(o, LANES)] = jnp.where(m, w16, 0.0)

    def scale_pass(b):
        for g in range(CH // LANES):
            w16 = wbuf[b, pl.ds(g * LANES, LANES)]
            for l in range(LANES):
                e = g * LANES + l
                w = w16[l]
                for dd in range(HID // LANES):
                    sl = pl.ds(dd * LANES, LANES)
                    rows2[b, e, sl] = rows2[b, e, sl] * w

    fire_pack(0, 0)
    fire_pack(1, 1)

    def half_step(i, b):
        @pl.when(jnp.logical_and(i >= 2, i <= NCHUNK + 1))
        def _():
            pltpu.make_async_copy(
                rows2.at[b], acc_sh.at[pl.ds(0, CH)], sem_s[b]).wait()

        @pl.when(i < NCHUNK)
        def _():
            pltpu.make_async_copy(
                pack_hbm.at[pl.ds((rbase + i) * REC, REC)], packb.at[b],
                sem_p[b]).wait()
            vec_pass(b)
            pltpu.async_copy(x_hbm.at[gsrc2.at[b]], rows2.at[b], sem_g[b])

        nb = 1 - b

        @pl.when(jnp.logical_and(i >= 1, i <= NCHUNK))
        def _():
            pltpu.make_async_copy(
                x_hbm.at[gsrc2.at[nb]], rows2.at[nb], sem_g[nb]).wait()
            scale_pass(nb)
            pltpu.async_copy(rows2.at[nb], acc_sh.at[lidx2.at[nb]],
                             sem_s[nb], add=True)

        @pl.when(i + 2 < NCHUNK)
        def _():
            fire_pack(i + 2, b)

    def step(g, carry):
        half_step(2 * g, 0)
        half_step(2 * g + 1, 1)
        return carry

    lax.fori_loop(0, (NCHUNK + 2) // 2, step, 0)

    plsc.subcore_barrier()
    pltpu.sync_copy(acc_sh.at[pl.ds(s * RPS, RPS)],
                    out_hbm.at[pl.ds(c * ACC_R + s * RPS, RPS)])


_spmm_call = pl.kernel(
    _spmm_body,
    out_type=jax.ShapeDtypeStruct((XROWS, HID), jnp.float32),
    mesh=plsc.VectorSubcoreMesh(core_axis_name="c", subcore_axis_name="s",
                                num_cores=NC, num_subcores=NS),
    scratch_types=[
        pltpu.VMEM_SHARED((ACC_R, HID), jnp.float32),
        pltpu.VMEM((2, REC), jnp.int32),
        pltpu.VMEM((2, CH), jnp.int32),
        pltpu.VMEM((2, CH), jnp.int32),
        pltpu.VMEM((2, CH), jnp.float32),
        pltpu.VMEM((2, CH, HID), jnp.float32),
        pltpu.SemaphoreType.DMA,
        pltpu.SemaphoreType.DMA,
        pltpu.SemaphoreType.DMA,
        pltpu.SemaphoreType.DMA,
        pltpu.SemaphoreType.DMA,
        pltpu.SemaphoreType.DMA,
    ],
    compiler_params=pltpu.CompilerParams(use_tc_tiling_on_sc=False,
                                         needs_layout_passes=False),
)


CG = 128
NIDX = NUM_ITEMS * HIST
NCH_G = 123
NW = NC * NS
GROWS = NW * NCH_G * CG


def _gather_body(idx_hbm, kv_hbm, out_hbm, idxb, rows2,
                 sem_p0, sem_p1, sem_g0, sem_g1, sem_w0, sem_w1):
    c = lax.axis_index("c")
    s = lax.axis_index("s")
    w = s * NC + c
    sem_p = (sem_p0, sem_p1)
    sem_g = (sem_g0, sem_g1)
    sem_w = (sem_w0, sem_w1)
    cbase = w * NCH_G

    def fire_idx(i, b):
        pltpu.async_copy(idx_hbm.at[pl.ds((cbase + i) * CG, CG)],
                         idxb.at[b], sem_p[b])

    fire_idx(0, 0)
    fire_idx(1, 1)

    def half_step(i, b):
        @pl.when(jnp.logical_and(i >= 2, i <= NCH_G + 1))
        def _():
            pltpu.make_async_copy(
                rows2.at[b], out_hbm.at[pl.ds(0, CG)], sem_w[b]).wait()

        @pl.when(i < NCH_G)
        def _():
            pltpu.make_async_copy(
                idx_hbm.at[pl.ds((cbase + i) * CG, CG)], idxb.at[b],
                sem_p[b]).wait()
            pltpu.async_copy(kv_hbm.at[idxb.at[b]], rows2.at[b], sem_g[b])

        nb = 1 - b

        @pl.when(jnp.logical_and(i >= 1, i <= NCH_G))
        def _():
            pltpu.make_async_copy(
                kv_hbm.at[idxb.at[nb]], rows2.at[nb], sem_g[nb]).wait()
            pltpu.async_copy(
                rows2.at[nb],
                out_hbm.at[pl.ds((cbase + i - 1) * CG, CG)], sem_w[nb])

        @pl.when(jnp.logical_and(i >= 1, i + 1 < NCH_G))
        def _():
            fire_idx(i + 1, nb)

    def step(g, carry):
        half_step(2 * g, 0)
        half_step(2 * g + 1, 1)
        return carry

    lax.fori_loop(0, (NCH_G + 3) // 2, step, 0)


_gather_call = pl.kernel(
    _gather_body,
    out_type=jax.ShapeDtypeStruct((GROWS, 2 * HID), jnp.float32),
    mesh=plsc.VectorSubcoreMesh(core_axis_name="c", subcore_axis_name="s",
                                num_cores=NC, num_subcores=NS),
    scratch_types=[
        pltpu.VMEM((2, CG), jnp.int32),
        pltpu.VMEM((2, CG, 2 * HID), jnp.float32),
        pltpu.SemaphoreType.DMA,
        pltpu.SemaphoreType.DMA,
        pltpu.SemaphoreType.DMA,
        pltpu.SemaphoreType.DMA,
        pltpu.SemaphoreType.DMA,
        pltpu.SemaphoreType.DMA,
    ],
)


B1 = 200
B2 = 200


def _feat_kernel(img_ref, txt_ref, w1ih_ref, w1il_ref, w1th_ref, w1tl_ref,
                 w2_ref, o_ref):
    x = img_ref[...]
    xh = x.astype(jnp.bfloat16)
    xl = (x - xh.astype(jnp.float32)).astype(jnp.bfloat16)
    t = txt_ref[...]
    th = t.astype(jnp.bfloat16)
    tl = (t - th.astype(jnp.float32)).astype(jnp.bfloat16)

    def bdot(a, b):
        return jnp.dot(a, b, preferred_element_type=jnp.float32)

    h = (bdot(xh, w1ih_ref[...]) + bdot(xh, w1il_ref[...])
         + bdot(xl, w1ih_ref[...])
         + bdot(th, w1th_ref[...]) + bdot(th, w1tl_ref[...])
         + bdot(tl, w1th_ref[...]))
    o_ref[...] = jnp.dot(h, w2_ref[...], preferred_element_type=jnp.float32,
                         precision=lax.Precision.HIGHEST)


def _feat_call(image_w, text_w, W_mul1, W_mul2):
    w1i = W_mul1[:, :4096].T
    w1t = W_mul1[:, 4096:].T
    w1ih = w1i.astype(jnp.bfloat16)
    w1il = (w1i - w1ih.astype(jnp.float32)).astype(jnp.bfloat16)
    w1th = w1t.astype(jnp.bfloat16)
    w1tl = (w1t - w1th.astype(jnp.float32)).astype(jnp.bfloat16)
    w2 = W_mul2.T
    return pl.pallas_call(
        _feat_kernel,
        grid=(NUM_ITEMS // B1,),
        in_specs=[
            pl.BlockSpec((B1, 4096), lambda i: (i, 0)),
            pl.BlockSpec((B1, 384), lambda i: (i, 0)),
            pl.BlockSpec((4096, 256), lambda i: (0, 0)),
            pl.BlockSpec((4096, 256), lambda i: (0, 0)),
            pl.BlockSpec((384, 256), lambda i: (0, 0)),
            pl.BlockSpec((384, 256), lambda i: (0, 0)),
            pl.BlockSpec((256, HID), lambda i: (0, 0)),
        ],
        out_specs=pl.BlockSpec((B1, HID), lambda i: (i, 0)),
        out_shape=jax.ShapeDtypeStruct((NUM_ITEMS, HID), jnp.float32),
    )(image_w, text_w, w1ih, w1il, w1th, w1tl, w2)


def _att_kernel(kv_ref, item_ref, qw_ref, bm_ref, ow_ref, itea_ref, o_ref):
    kv = kv_ref[...]
    k = kv[:, :HID]
    v = kv[:, HID:]
    item = item_ref[...]
    q = jnp.dot(item, qw_ref[...], preferred_element_type=jnp.float32,
                precision=lax.Precision.HIGHEST)
    k3 = k.reshape(B2, HIST, HID)
    p = q[:, None, :] * k3
    sb = jnp.dot(p.reshape(B2 * HIST, HID), bm_ref[...],
                 preferred_element_type=jnp.float32,
                 precision=lax.Precision.HIGHEST) * 0.125
    s3 = sb.reshape(B2, HIST, HID)
    mx = jnp.max(s3, axis=1, keepdims=True)
    ex = jnp.exp(s3 - mx)
    attb = ex / jnp.sum(ex, axis=1, keepdims=True)
    res = jnp.sum(attb * v.reshape(B2, HIST, HID), axis=1)
    oneh = jnp.dot(res, ow_ref[...], preferred_element_type=jnp.float32,
                   precision=lax.Precision.HIGHEST)
    o_ref[...] = item * itea_ref[...] + oneh


def _att_call(ph_kv, item_emb, Q, W_onehop, itea):
    bm = (jnp.arange(HID, dtype=jnp.int32)[:, None] // ATT
          == jnp.arange(HID, dtype=jnp.int32)[None, :] // ATT
          ).astype(jnp.float32)
    return pl.pallas_call(
        _att_kernel,
        grid=(NUM_ITEMS // B2,),
        in_specs=[
            pl.BlockSpec((B2 * HIST, 2 * HID), lambda i: (i, 0)),
            pl.BlockSpec((B2, HID), lambda i: (i, 0)),
            pl.BlockSpec((HID, HID), lambda i: (0, 0)),
            pl.BlockSpec((HID, HID), lambda i: (0, 0)),
            pl.BlockSpec((HID, HID), lambda i: (0, 0)),
            pl.BlockSpec((B2, HID), lambda i: (i, 0)),
        ],
        out_specs=pl.BlockSpec((B2, HID), lambda i: (i, 0)),
        out_shape=jax.ShapeDtypeStruct((NUM_ITEMS, HID), jnp.float32),
    )(ph_kv, item_emb, Q, bm, W_onehop.T, itea)


def _mean4_kernel(a_ref, b_ref, c_ref, d_ref, o_ref):
    o_ref[...] = 0.25 * (a_ref[...] + b_ref[...] + c_ref[...] + d_ref[...])


def _mean4_half(e0, e1, e2, e3, block_off):
    blk = 200
    in_spec = pl.BlockSpec((blk, HID), lambda i: (i + block_off, 0))
    out_spec = pl.BlockSpec((blk, HID), lambda i: (i, 0))
    return pl.pallas_call(
        _mean4_kernel,
        grid=(NUM_USERS // blk,),
        in_specs=[in_spec] * 4,
        out_specs=out_spec,
        out_shape=jax.ShapeDtypeStruct((NUM_USERS, HID), jnp.float32),
    )(e0, e1, e2, e3)


def kernel(photo_one_hop, user_emb, item_emb, image_w, text_w, Q, K, V,
           W_onehop, W_mul1, W_mul2, edge_index, edge_weight):
    kv = jnp.concatenate(
        [jnp.dot(user_emb, K, precision=lax.Precision.HIGHEST),
         jnp.dot(user_emb, V, precision=lax.Precision.HIGHEST)],
        axis=1)
    pad_g = GROWS - NIDX
    gfill = (jnp.arange(pad_g, dtype=jnp.int32) * 13) % NUM_USERS
    gidx = jnp.concatenate([photo_one_hop.reshape(-1), gfill])
    ph_kv = _gather_call(gidx, kv)

    itea_fea_emb = _feat_call(image_w, text_w, W_mul1, W_mul2)
    all_items = _att_call(ph_kv, item_emb, Q, W_onehop, itea_fea_emb)

    zpad = jnp.zeros((PAD, HID), jnp.float32)
    x = jnp.concatenate([user_emb, zpad, all_items, zpad], axis=0)

    pad_n = EDGES_P - N_EDGES
    fill = (jnp.arange(pad_n, dtype=jnp.int32) * 37) % N_NODES
    dst_p = jnp.concatenate([edge_index[0], fill])
    src_p = jnp.concatenate([edge_index[1], fill])
    w_p = jnp.concatenate([edge_weight, jnp.zeros((pad_n,), jnp.float32)])
    w_bits = jax.lax.bitcast_convert_type(w_p, jnp.int32)
    pack = jnp.stack([dst_p.reshape(-1, CH), src_p.reshape(-1, CH),
                      w_bits.reshape(-1, CH)], axis=1).reshape(-1)
    zeros_acc = jnp.zeros((ACC_R, HID), jnp.float32)

    embs = [x]
    for _ in range(N_LAYERS):
        x = _spmm_call(pack, x, zeros_acc)
        embs.append(x)

    users_emb = _mean4_half(embs[0], embs[1], embs[2], embs[3], 0)
    items_emb = _mean4_half(embs[0], embs[1], embs[2], embs[3], ACC_R // 200)
    return users_emb, items_emb


# speedup vs baseline: 5.4136x; 1.0666x over previous
import functools

import jax
import jax.numpy as jnp
from jax import lax
from jax.experimental import pallas as pl
from jax.experimental.pallas import tpu as pltpu
from jax.experimental.pallas import tpu_sc as plsc

NUM_USERS = 25000
NUM_ITEMS = 25000
HID = 64
N_LAYERS = 3
HIST = 20
N_NODES = NUM_USERS + NUM_ITEMS
N_EDGES = 800000
NH = 4
ATT = 16

NC = 2
NS = 16
LANES = 16

HALF = 25000
ACC_R = 25600
PAD = ACC_R - HALF
RPS = ACC_R // NS
XROWS = NC * ACC_R

CH = 128
NCHUNK = 392
EPS_P = NCHUNK * CH
EDGES_P = NS * EPS_P
REC = 3 * CH


def _spmm_body(pack_hbm, x_hbm, zeros_hbm, out_hbm,
               acc_sh, packb, gsrc2, lidx2, wbuf, rows2,
               sem_p0, sem_p1, sem_g0, sem_g1, sem_s0, sem_s1):
    c = lax.axis_index("c")
    s = lax.axis_index("s")
    half_base = c * HALF
    sem_p = (sem_p0, sem_p1)
    sem_g = (sem_g0, sem_g1)
    sem_s = (sem_s0, sem_s1)

    pltpu.sync_copy(zeros_hbm.at[pl.ds(s * RPS, RPS)],
                    acc_sh.at[pl.ds(s * RPS, RPS)])
    plsc.subcore_barrier()

    rbase = s * NCHUNK

    def fire_pack(i, b):
        pltpu.async_copy(pack_hbm.at[pl.ds((rbase + i) * REC, REC)],
                         packb.at[b], sem_p[b])

    def vec_pass(b):
        for g in range(CH // LANES):
            o = g * LANES
            d16 = packb[b, pl.ds(o, LANES)]
            s16 = packb[b, pl.ds(CH + o, LANES)]
            w16 = plsc.bitcast(packb[b, pl.ds(2 * CH + o, LANES)],
                               jnp.float32)
            loc = d16 - half_base
            m = (loc >= 0) & (loc < HALF)
            gsrc2[b, pl.ds(o, LANES)] = s16 + jnp.where(
                s16 >= HALF, jnp.int32(PAD), jnp.int32(0))
            lidx2[b, pl.ds(o, LANES)] = jnp.where(m, loc, loc & 8191)
            wbuf[b, pl.ds(o, LANES)] = jnp.where(m, w16, 0.0)

    def scale_pass(b):
        for g in range(CH // LANES):
            w16 = wbuf[b, pl.ds(g * LANES, LANES)]
            for l in range(LANES):
                e = g * LANES + l
                w = w16[l]
                for dd in range(HID // LANES):
                    sl = pl.ds(dd * LANES, LANES)
                    rows2[b, e, sl] = rows2[b, e, sl] * w

    fire_pack(0, 0)
    fire_pack(1, 1)

    def half_step(i, b):
        @pl.when(jnp.logical_and(i >= 2, i <= NCHUNK + 1))
        def _():
            pltpu.make_async_copy(
                rows2.at[b], acc_sh.at[pl.ds(0, CH)], sem_s[b]).wait()

        @pl.when(i < NCHUNK)
        def _():
            pltpu.make_async_copy(
                pack_hbm.at[pl.ds((rbase + i) * REC, REC)], packb.at[b],
                sem_p[b]).wait()
            vec_pass(b)
            pltpu.async_copy(x_hbm.at[gsrc2.at[b]], rows2.at[b], sem_g[b])

        nb = 1 - b

        @pl.when(jnp.logical_and(i >= 1, i <= NCHUNK))
        def _():
            pltpu.make_async_copy(
                x_hbm.at[gsrc2.at[nb]], rows2.at[nb], sem_g[nb]).wait()
            scale_pass(nb)
            pltpu.async_copy(rows2.at[nb], acc_sh.at[lidx2.at[nb]],
                             sem_s[nb], add=True)

        @pl.when(i + 2 < NCHUNK)
        def _():
            fire_pack(i + 2, b)

    def step(g, carry):
        half_step(2 * g, 0)
        half_step(2 * g + 1, 1)
        return carry

    lax.fori_loop(0, (NCHUNK + 2) // 2, step, 0)

    plsc.subcore_barrier()
    pltpu.sync_copy(acc_sh.at[pl.ds(s * RPS, RPS)],
                    out_hbm.at[pl.ds(c * ACC_R + s * RPS, RPS)])


_spmm_call = pl.kernel(
    _spmm_body,
    out_type=jax.ShapeDtypeStruct((XROWS, HID), jnp.float32),
    mesh=plsc.VectorSubcoreMesh(core_axis_name="c", subcore_axis_name="s",
                                num_cores=NC, num_subcores=NS),
    scratch_types=[
        pltpu.VMEM_SHARED((ACC_R, HID), jnp.float32),
        pltpu.VMEM((2, REC), jnp.int32),
        pltpu.VMEM((2, CH), jnp.int32),
        pltpu.VMEM((2, CH), jnp.int32),
        pltpu.VMEM((2, CH), jnp.float32),
        pltpu.VMEM((2, CH, HID), jnp.float32),
        pltpu.SemaphoreType.DMA,
        pltpu.SemaphoreType.DMA,
        pltpu.SemaphoreType.DMA,
        pltpu.SemaphoreType.DMA,
        pltpu.SemaphoreType.DMA,
        pltpu.SemaphoreType.DMA,
    ],
    compiler_params=pltpu.CompilerParams(use_tc_tiling_on_sc=False,
                                         needs_layout_passes=False),
)


CG = 128
NIDX = NUM_ITEMS * HIST
NCH_G = 123
NW = NC * NS
GROWS = NW * NCH_G * CG


def _gather_body(idx_hbm, kv_hbm, out_hbm, idxb, rows2,
                 sem_p0, sem_p1, sem_g0, sem_g1, sem_w0, sem_w1):
    c = lax.axis_index("c")
    s = lax.axis_index("s")
    w = s * NC + c
    sem_p = (sem_p0, sem_p1)
    sem_g = (sem_g0, sem_g1)
    sem_w = (sem_w0, sem_w1)
    cbase = w * NCH_G

    def fire_idx(i, b):
        pltpu.async_copy(idx_hbm.at[pl.ds((cbase + i) * CG, CG)],
                         idxb.at[b], sem_p[b])

    fire_idx(0, 0)
    fire_idx(1, 1)

    def half_step(i, b):
        @pl.when(jnp.logical_and(i >= 2, i <= NCH_G + 1))
        def _():
            pltpu.make_async_copy(
                rows2.at[b], out_hbm.at[pl.ds(0, CG)], sem_w[b]).wait()

        @pl.when(i < NCH_G)
        def _():
            pltpu.make_async_copy(
                idx_hbm.at[pl.ds((cbase + i) * CG, CG)], idxb.at[b],
                sem_p[b]).wait()
            pltpu.async_copy(kv_hbm.at[idxb.at[b]], rows2.at[b], sem_g[b])

        nb = 1 - b

        @pl.when(jnp.logical_and(i >= 1, i <= NCH_G))
        def _():
            pltpu.make_async_copy(
                kv_hbm.at[idxb.at[nb]], rows2.at[nb], sem_g[nb]).wait()
            pltpu.async_copy(
                rows2.at[nb],
                out_hbm.at[pl.ds((cbase + i - 1) * CG, CG)], sem_w[nb])

        @pl.when(jnp.logical_and(i >= 1, i + 1 < NCH_G))
        def _():
            fire_idx(i + 1, nb)

    def step(g, carry):
        half_step(2 * g, 0)
        half_step(2 * g + 1, 1)
        return carry

    lax.fori_loop(0, (NCH_G + 3) // 2, step, 0)


_gather_call = pl.kernel(
    _gather_body,
    out_type=jax.ShapeDtypeStruct((GROWS, 2 * HID), jnp.float32),
    mesh=plsc.VectorSubcoreMesh(core_axis_name="c", subcore_axis_name="s",
                                num_cores=NC, num_subcores=NS),
    scratch_types=[
        pltpu.VMEM((2, CG), jnp.int32),
        pltpu.VMEM((2, CG, 2 * HID), jnp.float32),
        pltpu.SemaphoreType.DMA,
        pltpu.SemaphoreType.DMA,
        pltpu.SemaphoreType.DMA,
        pltpu.SemaphoreType.DMA,
        pltpu.SemaphoreType.DMA,
        pltpu.SemaphoreType.DMA,
    ],
)


B1 = 200
B2 = 200


def _feat_kernel(img_ref, txt_ref, w1ih_ref, w1il_ref, w1th_ref, w1tl_ref,
                 w2_ref, o_ref):
    x = img_ref[...]
    xh = x.astype(jnp.bfloat16)
    xl = (x - xh.astype(jnp.float32)).astype(jnp.bfloat16)
    t = txt_ref[...]
    th = t.astype(jnp.bfloat16)
    tl = (t - th.astype(jnp.float32)).astype(jnp.bfloat16)

    def bdot(a, b):
        return jnp.dot(a, b, preferred_element_type=jnp.float32)

    h = (bdot(xh, w1ih_ref[...]) + bdot(xh, w1il_ref[...])
         + bdot(xl, w1ih_ref[...])
         + bdot(th, w1th_ref[...]) + bdot(th, w1tl_ref[...])
         + bdot(tl, w1th_ref[...]))
    o_ref[...] = jnp.dot(h, w2_ref[...], preferred_element_type=jnp.float32,
                         precision=lax.Precision.HIGHEST)


def _feat_call(image_w, text_w, W_mul1, W_mul2):
    w1i = W_mul1[:, :4096].T
    w1t = W_mul1[:, 4096:].T
    w1ih = w1i.astype(jnp.bfloat16)
    w1il = (w1i - w1ih.astype(jnp.float32)).astype(jnp.bfloat16)
    w1th = w1t.astype(jnp.bfloat16)
    w1tl = (w1t - w1th.astype(jnp.float32)).astype(jnp.bfloat16)
    w2 = W_mul2.T
    return pl.pallas_call(
        _feat_kernel,
        grid=(NUM_ITEMS // B1,),
        in_specs=[
            pl.BlockSpec((B1, 4096), lambda i: (i, 0)),
            pl.BlockSpec((B1, 384), lambda i: (i, 0)),
            pl.BlockSpec((4096, 256), lambda i: (0, 0)),
            pl.BlockSpec((4096, 256), lambda i: (0, 0)),
            pl.BlockSpec((384, 256), lambda i: (0, 0)),
            pl.BlockSpec((384, 256), lambda i: (0, 0)),
            pl.BlockSpec((256, HID), lambda i: (0, 0)),
        ],
        out_specs=pl.BlockSpec((B1, HID), lambda i: (i, 0)),
        out_shape=jax.ShapeDtypeStruct((NUM_ITEMS, HID), jnp.float32),
    )(image_w, text_w, w1ih, w1il, w1th, w1tl, w2)


def _att_kernel(kv_ref, item_ref, qw_ref, bm_ref, ow_ref, itea_ref, o_ref):
    kv = kv_ref[...]
    k = kv[:, :HID]
    v = kv[:, HID:]
    item = item_ref[...]
    q = jnp.dot(item, qw_ref[...], preferred_element_type=jnp.float32,
                precision=lax.Precision.HIGHEST)
    k3 = k.reshape(B2, HIST, HID)
    p = q[:, None, :] * k3
    p2 = p.reshape(B2 * HIST, HID)
    ph = p2.astype(jnp.bfloat16)
    pl_ = (p2 - ph.astype(jnp.float32)).astype(jnp.bfloat16)
    bmh = bm_ref[...]
    sb = (jnp.dot(ph, bmh, preferred_element_type=jnp.float32)
          + jnp.dot(pl_, bmh, preferred_element_type=jnp.float32)) * 0.125
    s3 = sb.reshape(B2, HIST, HID)
    mx = jnp.max(s3, axis=1, keepdims=True)
    ex = jnp.exp(s3 - mx)
    den = jnp.sum(ex, axis=1)
    num = jnp.sum(ex * v.reshape(B2, HIST, HID), axis=1)
    res = num / den
    oneh = jnp.dot(res, ow_ref[...], preferred_element_type=jnp.float32,
                   precision=lax.Precision.HIGHEST)
    o_ref[...] = item * itea_ref[...] + oneh


def _att_call(ph_kv, item_emb, Q, W_onehop, itea):
    bm = (jnp.arange(HID, dtype=jnp.int32)[:, None] // ATT
          == jnp.arange(HID, dtype=jnp.int32)[None, :] // ATT
          ).astype(jnp.bfloat16)
    return pl.pallas_call(
        _att_kernel,
        grid=(NUM_ITEMS // B2,),
        in_specs=[
            pl.BlockSpec((B2 * HIST, 2 * HID), lambda i: (i, 0)),
            pl.BlockSpec((B2, HID), lambda i: (i, 0)),
            pl.BlockSpec((HID, HID), lambda i: (0, 0)),
            pl.BlockSpec((HID, HID), lambda i: (0, 0)),
            pl.BlockSpec((HID, HID), lambda i: (0, 0)),
            pl.BlockSpec((B2, HID), lambda i: (i, 0)),
        ],
        out_specs=pl.BlockSpec((B2, HID), lambda i: (i, 0)),
        out_shape=jax.ShapeDtypeStruct((NUM_ITEMS, HID), jnp.float32),
    )(ph_kv, item_emb, Q, bm, W_onehop.T, itea)


def _mean4_kernel(a_ref, b_ref, c_ref, d_ref, o_ref):
    o_ref[...] = 0.25 * (a_ref[...] + b_ref[...] + c_ref[...] + d_ref[...])


def _mean4_half(e0, e1, e2, e3, block_off):
    blk = 200
    in_spec = pl.BlockSpec((blk, HID), lambda i: (i + block_off, 0))
    out_spec = pl.BlockSpec((blk, HID), lambda i: (i, 0))
    return pl.pallas_call(
        _mean4_kernel,
        grid=(NUM_USERS // blk,),
        in_specs=[in_spec] * 4,
        out_specs=out_spec,
        out_shape=jax.ShapeDtypeStruct((NUM_USERS, HID), jnp.float32),
    )(e0, e1, e2, e3)


def kernel(photo_one_hop, user_emb, item_emb, image_w, text_w, Q, K, V,
           W_onehop, W_mul1, W_mul2, edge_index, edge_weight):
    kv = jnp.concatenate(
        [jnp.dot(user_emb, K, precision=lax.Precision.HIGHEST),
         jnp.dot(user_emb, V, precision=lax.Precision.HIGHEST)],
        axis=1)
    pad_g = GROWS - NIDX
    gfill = (jnp.arange(pad_g, dtype=jnp.int32) * 13) % NUM_USERS
    gidx = jnp.concatenate([photo_one_hop.reshape(-1), gfill])
    ph_kv = _gather_call(gidx, kv)

    itea_fea_emb = _feat_call(image_w, text_w, W_mul1, W_mul2)
    all_items = _att_call(ph_kv, item_emb, Q, W_onehop, itea_fea_emb)

    zpad = jnp.zeros((PAD, HID), jnp.float32)
    x = jnp.concatenate([user_emb, zpad, all_items, zpad], axis=0)

    pad_n = EDGES_P - N_EDGES
    fill = (jnp.arange(pad_n, dtype=jnp.int32) * 37) % N_NODES
    dst_p = jnp.concatenate([edge_index[0], fill])
    src_p = jnp.concatenate([edge_index[1], fill])
    w_p = jnp.concatenate([edge_weight, jnp.zeros((pad_n,), jnp.float32)])
    w_bits = jax.lax.bitcast_convert_type(w_p, jnp.int32)
    pack = jnp.stack([dst_p.reshape(-1, CH), src_p.reshape(-1, CH),
                      w_bits.reshape(-1, CH)], axis=1).reshape(-1)
    zeros_acc = jnp.zeros((ACC_R, HID), jnp.float32)

    embs = [x]
    for _ in range(N_LAYERS):
        x = _spmm_call(pack, x, zeros_acc)
        embs.append(x)

    users_emb = _mean4_half(embs[0], embs[1], embs[2], embs[3], 0)
    items_emb = _mean4_half(embs[0], embs[1], embs[2], embs[3], ACC_R // 200)
    return users_emb, items_emb


# speedup vs baseline: 5.7246x; 1.0575x over previous
import functools

import jax
import jax.numpy as jnp
from jax import lax
from jax.experimental import pallas as pl
from jax.experimental.pallas import tpu as pltpu
from jax.experimental.pallas import tpu_sc as plsc

NUM_USERS = 25000
NUM_ITEMS = 25000
HID = 64
N_LAYERS = 3
HIST = 20
N_NODES = NUM_USERS + NUM_ITEMS
N_EDGES = 800000
NH = 4
ATT = 16

NC = 2
NS = 16
LANES = 16

HALF = 25000
ACC_R = 25600
PAD = ACC_R - HALF
RPS = ACC_R // NS
XROWS = NC * ACC_R

CH = 128
NCHUNK = 392
EPS_P = NCHUNK * CH
EDGES_P = NS * EPS_P
REC = 3 * CH


def _spmm_body(pack_hbm, x_hbm, zeros_hbm, out_hbm,
               acc_sh, packb, gsrc2, lidx2, wbuf, rows2,
               sem_p0, sem_p1, sem_g0, sem_g1, sem_s0, sem_s1):
    c = lax.axis_index("c")
    s = lax.axis_index("s")
    half_base = c * HALF
    sem_p = (sem_p0, sem_p1)
    sem_g = (sem_g0, sem_g1)
    sem_s = (sem_s0, sem_s1)

    pltpu.sync_copy(zeros_hbm.at[pl.ds(s * RPS, RPS)],
                    acc_sh.at[pl.ds(s * RPS, RPS)])
    plsc.subcore_barrier()

    rbase = s * NCHUNK

    def fire_pack(i, b):
        pltpu.async_copy(pack_hbm.at[pl.ds((rbase + i) * REC, REC)],
                         packb.at[b], sem_p[b])

    def vec_pass(b):
        for g in range(CH // LANES):
            o = g * LANES
            d16 = packb[b, pl.ds(o, LANES)]
            s16 = packb[b, pl.ds(CH + o, LANES)]
            w16 = plsc.bitcast(packb[b, pl.ds(2 * CH + o, LANES)],
                               jnp.float32)
            loc = d16 - half_base
            m = (loc >= 0) & (loc < HALF)
            gsrc2[b, pl.ds(o, LANES)] = s16 + jnp.where(
                s16 >= HALF, jnp.int32(PAD), jnp.int32(0))
            lidx2[b, pl.ds(o, LANES)] = jnp.where(m, loc, loc & 8191)
            wbuf[b, pl.ds(o, LANES)] = jnp.where(m, w16, 0.0)

    def scale_pass(b):
        for g in range(CH // LANES):
            w16 = wbuf[b, pl.ds(g * LANES, LANES)]
            for l in range(LANES):
                e = g * LANES + l
                w = w16[l]
                for dd in range(HID // LANES):
                    sl = pl.ds(dd * LANES, LANES)
                    rows2[b, e, sl] = rows2[b, e, sl] * w

    fire_pack(0, 0)
    fire_pack(1, 1)

    def half_step(i, b):
        @pl.when(jnp.logical_and(i >= 2, i <= NCHUNK + 1))
        def _():
            pltpu.make_async_copy(
                rows2.at[b], acc_sh.at[pl.ds(0, CH)], sem_s[b]).wait()

        @pl.when(i < NCHUNK)
        def _():
            pltpu.make_async_copy(
                pack_hbm.at[pl.ds((rbase + i) * REC, REC)], packb.at[b],
                sem_p[b]).wait()
            vec_pass(b)
            pltpu.async_copy(x_hbm.at[gsrc2.at[b]], rows2.at[b], sem_g[b])

        nb = 1 - b

        @pl.when(jnp.logical_and(i >= 1, i <= NCHUNK))
        def _():
            pltpu.make_async_copy(
                x_hbm.at[gsrc2.at[nb]], rows2.at[nb], sem_g[nb]).wait()
            scale_pass(nb)
            pltpu.async_copy(rows2.at[nb], acc_sh.at[lidx2.at[nb]],
                             sem_s[nb], add=True)

        @pl.when(i + 2 < NCHUNK)
        def _():
            fire_pack(i + 2, b)

    def step(g, carry):
        half_step(2 * g, 0)
        half_step(2 * g + 1, 1)
        return carry

    lax.fori_loop(0, (NCHUNK + 2) // 2, step, 0)

    plsc.subcore_barrier()
    pltpu.sync_copy(acc_sh.at[pl.ds(s * RPS, RPS)],
                    out_hbm.at[pl.ds(c * ACC_R + s * RPS, RPS)])


_spmm_call = pl.kernel(
    _spmm_body,
    out_type=jax.ShapeDtypeStruct((XROWS, HID), jnp.float32),
    mesh=plsc.VectorSubcoreMesh(core_axis_name="c", subcore_axis_name="s",
                                num_cores=NC, num_subcores=NS),
    scratch_types=[
        pltpu.VMEM_SHARED((ACC_R, HID), jnp.float32),
        pltpu.VMEM((2, REC), jnp.int32),
        pltpu.VMEM((2, CH), jnp.int32),
        pltpu.VMEM((2, CH), jnp.int32),
        pltpu.VMEM((2, CH), jnp.float32),
        pltpu.VMEM((2, CH, HID), jnp.float32),
        pltpu.SemaphoreType.DMA,
        pltpu.SemaphoreType.DMA,
        pltpu.SemaphoreType.DMA,
        pltpu.SemaphoreType.DMA,
        pltpu.SemaphoreType.DMA,
        pltpu.SemaphoreType.DMA,
    ],
    compiler_params=pltpu.CompilerParams(use_tc_tiling_on_sc=False,
                                         needs_layout_passes=False),
)


CG = 128
NIDX = NUM_ITEMS * HIST
NCH_G = 123
NW = NC * NS
GROWS = NW * NCH_G * CG


def _gather_body(idx_hbm, kv_hbm, out_hbm, idxb, rows2,
                 sem_p0, sem_p1, sem_g0, sem_g1, sem_w0, sem_w1):
    c = lax.axis_index("c")
    s = lax.axis_index("s")
    w = s * NC + c
    sem_p = (sem_p0, sem_p1)
    sem_g = (sem_g0, sem_g1)
    sem_w = (sem_w0, sem_w1)
    cbase = w * NCH_G

    def fire_idx(i, b):
        pltpu.async_copy(idx_hbm.at[pl.ds((cbase + i) * CG, CG)],
                         idxb.at[b], sem_p[b])

    fire_idx(0, 0)
    fire_idx(1, 1)

    def half_step(i, b):
        @pl.when(jnp.logical_and(i >= 2, i <= NCH_G + 1))
        def _():
            pltpu.make_async_copy(
                rows2.at[b], out_hbm.at[pl.ds(0, CG)], sem_w[b]).wait()

        @pl.when(i < NCH_G)
        def _():
            pltpu.make_async_copy(
                idx_hbm.at[pl.ds((cbase + i) * CG, CG)], idxb.at[b],
                sem_p[b]).wait()
            pltpu.async_copy(kv_hbm.at[idxb.at[b]], rows2.at[b], sem_g[b])

        nb = 1 - b

        @pl.when(jnp.logical_and(i >= 1, i <= NCH_G))
        def _():
            pltpu.make_async_copy(
                kv_hbm.at[idxb.at[nb]], rows2.at[nb], sem_g[nb]).wait()
            pltpu.async_copy(
                rows2.at[nb],
                out_hbm.at[pl.ds((cbase + i - 1) * CG, CG)], sem_w[nb])

        @pl.when(jnp.logical_and(i >= 1, i + 1 < NCH_G))
        def _():
            fire_idx(i + 1, nb)

    def step(g, carry):
        half_step(2 * g, 0)
        half_step(2 * g + 1, 1)
        return carry

    lax.fori_loop(0, (NCH_G + 3) // 2, step, 0)


_gather_call = pl.kernel(
    _gather_body,
    out_type=jax.ShapeDtypeStruct((GROWS, 2 * HID), jnp.float32),
    mesh=plsc.VectorSubcoreMesh(core_axis_name="c", subcore_axis_name="s",
                                num_cores=NC, num_subcores=NS),
    scratch_types=[
        pltpu.VMEM((2, CG), jnp.int32),
        pltpu.VMEM((2, CG, 2 * HID), jnp.float32),
        pltpu.SemaphoreType.DMA,
        pltpu.SemaphoreType.DMA,
        pltpu.SemaphoreType.DMA,
        pltpu.SemaphoreType.DMA,
        pltpu.SemaphoreType.DMA,
        pltpu.SemaphoreType.DMA,
    ],
)


B1 = 200
B2 = 1000


def _feat_kernel(img_ref, txt_ref, w1ih_ref, w1il_ref, w1th_ref, w1tl_ref,
                 w2_ref, o_ref):
    x = img_ref[...]
    xh = x.astype(jnp.bfloat16)
    xl = (x - xh.astype(jnp.float32)).astype(jnp.bfloat16)
    t = txt_ref[...]
    th = t.astype(jnp.bfloat16)
    tl = (t - th.astype(jnp.float32)).astype(jnp.bfloat16)

    def bdot(a, b):
        return jnp.dot(a, b, preferred_element_type=jnp.float32)

    h = (bdot(xh, w1ih_ref[...]) + bdot(xh, w1il_ref[...])
         + bdot(xl, w1ih_ref[...])
         + bdot(th, w1th_ref[...]) + bdot(th, w1tl_ref[...])
         + bdot(tl, w1th_ref[...]))
    o_ref[...] = jnp.dot(h, w2_ref[...], preferred_element_type=jnp.float32,
                         precision=lax.Precision.HIGHEST)


def _feat_call(image_w, text_w, W_mul1, W_mul2):
    w1i = W_mul1[:, :4096].T
    w1t = W_mul1[:, 4096:].T
    w1ih = w1i.astype(jnp.bfloat16)
    w1il = (w1i - w1ih.astype(jnp.float32)).astype(jnp.bfloat16)
    w1th = w1t.astype(jnp.bfloat16)
    w1tl = (w1t - w1th.astype(jnp.float32)).astype(jnp.bfloat16)
    w2 = W_mul2.T
    return pl.pallas_call(
        _feat_kernel,
        grid=(NUM_ITEMS // B1,),
        in_specs=[
            pl.BlockSpec((B1, 4096), lambda i: (i, 0)),
            pl.BlockSpec((B1, 384), lambda i: (i, 0)),
            pl.BlockSpec((4096, 256), lambda i: (0, 0)),
            pl.BlockSpec((4096, 256), lambda i: (0, 0)),
            pl.BlockSpec((384, 256), lambda i: (0, 0)),
            pl.BlockSpec((384, 256), lambda i: (0, 0)),
            pl.BlockSpec((256, HID), lambda i: (0, 0)),
        ],
        out_specs=pl.BlockSpec((B1, HID), lambda i: (i, 0)),
        out_shape=jax.ShapeDtypeStruct((NUM_ITEMS, HID), jnp.float32),
    )(image_w, text_w, w1ih, w1il, w1th, w1tl, w2)


def _att_kernel(kv_ref, item_ref, qw_ref, bm_ref, ow_ref, itea_ref, o_ref):
    kv = kv_ref[...]
    k = kv[:, :HID]
    v = kv[:, HID:]
    item = item_ref[...]
    q = jnp.dot(item, qw_ref[...], preferred_element_type=jnp.float32,
                precision=lax.Precision.HIGHEST)
    k3 = k.reshape(B2, HIST, HID)
    p = q[:, None, :] * k3
    p2 = p.reshape(B2 * HIST, HID)
    ph = p2.astype(jnp.bfloat16)
    pl_ = (p2 - ph.astype(jnp.float32)).astype(jnp.bfloat16)
    bmh = bm_ref[...]
    sb = (jnp.dot(ph, bmh, preferred_element_type=jnp.float32)
          + jnp.dot(pl_, bmh, preferred_element_type=jnp.float32)) * 0.125
    s3 = sb.reshape(B2, HIST, HID)
    mx = jnp.max(s3, axis=1, keepdims=True)
    ex = jnp.exp(s3 - mx)
    den = jnp.sum(ex, axis=1)
    num = jnp.sum(ex * v.reshape(B2, HIST, HID), axis=1)
    res = num / den
    oneh = jnp.dot(res, ow_ref[...], preferred_element_type=jnp.float32,
                   precision=lax.Precision.HIGHEST)
    o_ref[...] = item * itea_ref[...] + oneh


def _att_call(ph_kv, item_emb, Q, W_onehop, itea):
    bm = (jnp.arange(HID, dtype=jnp.int32)[:, None] // ATT
          == jnp.arange(HID, dtype=jnp.int32)[None, :] // ATT
          ).astype(jnp.bfloat16)
    return pl.pallas_call(
        _att_kernel,
        grid=(NUM_ITEMS // B2,),
        in_specs=[
            pl.BlockSpec((B2 * HIST, 2 * HID), lambda i: (i, 0)),
            pl.BlockSpec((B2, HID), lambda i: (i, 0)),
            pl.BlockSpec((HID, HID), lambda i: (0, 0)),
            pl.BlockSpec((HID, HID), lambda i: (0, 0)),
            pl.BlockSpec((HID, HID), lambda i: (0, 0)),
            pl.BlockSpec((B2, HID), lambda i: (i, 0)),
        ],
        out_specs=pl.BlockSpec((B2, HID), lambda i: (i, 0)),
        out_shape=jax.ShapeDtypeStruct((NUM_ITEMS, HID), jnp.float32),
    )(ph_kv, item_emb, Q, bm, W_onehop.T, itea)


def _mean4_kernel(a_ref, b_ref, c_ref, d_ref, o_ref):
    o_ref[...] = 0.25 * (a_ref[...] + b_ref[...] + c_ref[...] + d_ref[...])


def _mean4_half(e0, e1, e2, e3, block_off):
    blk = 1600
    in_spec = pl.BlockSpec((blk, HID), lambda i: (i + block_off, 0))
    out_spec = pl.BlockSpec((blk, HID), lambda i: (i, 0))
    return pl.pallas_call(
        _mean4_kernel,
        grid=(ACC_R // blk,),
        in_specs=[in_spec] * 4,
        out_specs=out_spec,
        out_shape=jax.ShapeDtypeStruct((ACC_R, HID), jnp.float32),
    )(e0, e1, e2, e3)[:NUM_USERS]


def kernel(photo_one_hop, user_emb, item_emb, image_w, text_w, Q, K, V,
           W_onehop, W_mul1, W_mul2, edge_index, edge_weight):
    kv = jnp.concatenate(
        [jnp.dot(user_emb, K, precision=lax.Precision.HIGHEST),
         jnp.dot(user_emb, V, precision=lax.Precision.HIGHEST)],
        axis=1)
    pad_g = GROWS - NIDX
    gfill = (jnp.arange(pad_g, dtype=jnp.int32) * 13) % NUM_USERS
    gidx = jnp.concatenate([photo_one_hop.reshape(-1), gfill])
    ph_kv = _gather_call(gidx, kv)

    itea_fea_emb = _feat_call(image_w, text_w, W_mul1, W_mul2)
    all_items = _att_call(ph_kv, item_emb, Q, W_onehop, itea_fea_emb)

    zpad = jnp.zeros((PAD, HID), jnp.float32)
    x = jnp.concatenate([user_emb, zpad, all_items, zpad], axis=0)

    pad_n = EDGES_P - N_EDGES
    fill = (jnp.arange(pad_n, dtype=jnp.int32) * 37) % N_NODES
    dst_p = jnp.concatenate([edge_index[0], fill])
    src_p = jnp.concatenate([edge_index[1], fill])
    w_p = jnp.concatenate([edge_weight, jnp.zeros((pad_n,), jnp.float32)])
    w_bits = jax.lax.bitcast_convert_type(w_p, jnp.int32)
    pack = jnp.stack([dst_p.reshape(-1, CH), src_p.reshape(-1, CH),
                      w_bits.reshape(-1, CH)], axis=1).reshape(-1)
    zeros_acc = jnp.zeros((ACC_R, HID), jnp.float32)

    embs = [x]
    for _ in range(N_LAYERS):
        x = _spmm_call(pack, x, zeros_acc)
        embs.append(x)

    users_emb = _mean4_half(embs[0], embs[1], embs[2], embs[3], 0)
    items_emb = _mean4_half(embs[0], embs[1], embs[2], embs[3],
                            ACC_R // 1600)
    return users_emb, items_emb
